# baseline (speedup 1.0000x reference)
import functools

import jax
import jax.numpy as jnp
from jax import lax
from jax.experimental import pallas as pl
from jax.experimental.pallas import tpu as pltpu


@functools.lru_cache(maxsize=None)
def _make_gather2(B, D):
    from jax.experimental.pallas import tpu_sc as plsc

    info = plsc.get_sparse_core_info()
    nc, ns = info.num_cores, info.num_subcores
    nw = nc * ns
    assert B % (8 * nw) == 0
    bpw = B // nw
    mesh = plsc.VectorSubcoreMesh(core_axis_name="c", subcore_axis_name="s")

    @functools.partial(
        pl.kernel,
        mesh=mesh,
        compiler_params=pltpu.CompilerParams(use_tc_tiling_on_sc=False),
        out_type=[
            jax.ShapeDtypeStruct((B, D), jnp.float32),
            jax.ShapeDtypeStruct((B, D), jnp.float32),
        ],
        scratch_types=[
            pltpu.VMEM((bpw,), jnp.int32),
            pltpu.VMEM((bpw,), jnp.int32),
            pltpu.VMEM((bpw, D), jnp.float32),
            pltpu.VMEM((bpw, D), jnp.float32),
            pltpu.SemaphoreType.DMA,
            pltpu.SemaphoreType.DMA,
        ],
    )
    def gather2(uemb, uids, bemb, bids, out_u, out_b,
                uidx_v, bidx_v, urows_v, brows_v, sem_u, sem_b):
        wid = lax.axis_index("s") * nc + lax.axis_index("c")
        base = wid * bpw
        pltpu.sync_copy(uids.at[pl.ds(base, bpw)], uidx_v)
        pltpu.sync_copy(bids.at[pl.ds(base, bpw)], bidx_v)
        cu = pltpu.async_copy(uemb.at[uidx_v], urows_v, sem_u)
        cb = pltpu.async_copy(bemb.at[bidx_v], brows_v, sem_b)
        cu.wait()
        pltpu.sync_copy(urows_v, out_u.at[pl.ds(base, bpw)])
        cb.wait()
        pltpu.sync_copy(brows_v, out_b.at[pl.ds(base, bpw)])

    return gather2


def _mlp_body(xu_ref, xb_ref, w1u_ref, w1b_ref, b1_ref, w2_ref, b2_ref,
              w3_ref, b3_ref, out_ref):
    dn = (((1,), (1,)), ((), ()))
    h = lax.dot_general(xu_ref[...], w1u_ref[...], dn,
                        preferred_element_type=jnp.float32)
    h = h + lax.dot_general(xb_ref[...], w1b_ref[...], dn,
                            preferred_element_type=jnp.float32)
    h = jnp.maximum(h + b1_ref[...], 0.0)
    h = lax.dot_general(h, w2_ref[...], dn, preferred_element_type=jnp.float32)
    h = jnp.maximum(h + b2_ref[...], 0.0)
    out = jnp.sum(h * w3_ref[...], axis=1, keepdims=True)
    out_ref[...] = out + b3_ref[0, 0]


def _mlp(xu, xb, W1u, W1b, b1, W2, b2, W3, b3, blk=2048):
    B, D = xu.shape
    H1 = W1u.shape[0]
    H2 = W2.shape[0]
    grid = (B // blk,)
    full = lambda shape: pl.BlockSpec(shape, lambda i: (0, 0))
    return pl.pallas_call(
        _mlp_body,
        grid=grid,
        in_specs=[
            pl.BlockSpec((blk, D), lambda i: (i, 0)),
            pl.BlockSpec((blk, D), lambda i: (i, 0)),
            full((H1, D)),
            full((H1, D)),
            full((1, H1)),
            full((H2, H1)),
            full((1, H2)),
            full((1, H2)),
            full((1, 1)),
        ],
        out_specs=pl.BlockSpec((blk, 1), lambda i: (i, 0)),
        out_shape=jax.ShapeDtypeStruct((B, 1), jnp.float32),
    )(xu, xb, W1u, W1b, b1, W2, b2, W3, b3)


def kernel(user_ids, book_ids, user_emb, book_emb, W1, b1, W2, b2, W3, b3):
    B = user_ids.shape[0]
    D = user_emb.shape[1]
    xu, xb = _make_gather2(B, D)(user_emb, user_ids.astype(jnp.int32),
                                 book_emb, book_ids.astype(jnp.int32))
    out = _mlp(xu, xb, W1[:, :D], W1[:, D:], b1.reshape(1, -1),
               W2, b2.reshape(1, -1), W3, b3.reshape(1, 1))
    return out.reshape(B)


# speedup vs baseline: 1.5749x; 1.5749x over previous
import functools

import jax
import jax.numpy as jnp
from jax import lax
from jax.experimental import pallas as pl
from jax.experimental.pallas import tpu as pltpu


@functools.lru_cache(maxsize=None)
def _make_gather2(B, D):
    from jax.experimental.pallas import tpu_sc as plsc

    info = plsc.get_sparse_core_info()
    nc, ns = info.num_cores, info.num_subcores
    nw = nc * ns
    assert B % (8 * nw) == 0
    bpw = B // nw
    ch = 16
    mesh = plsc.VectorSubcoreMesh(core_axis_name="c", subcore_axis_name="s")

    @functools.partial(
        pl.kernel,
        mesh=mesh,
        out_type=[
            jax.ShapeDtypeStruct((B, D), jnp.float32),
            jax.ShapeDtypeStruct((B, D), jnp.float32),
        ],
        scratch_types=[
            pltpu.VMEM((bpw,), jnp.int32),
            pltpu.VMEM((bpw, D), jnp.float32),
            pltpu.SemaphoreType.DMA,
        ],
    )
    def gather2(uemb, uids, bemb, bids, out_u, out_b, idx_v, rows_v, sem):
        wid = lax.axis_index("s") * nc + lax.axis_index("c")
        base = wid * bpw

        def one_table(emb, ids, out):
            pltpu.sync_copy(ids.at[pl.ds(base, bpw)], idx_v)

            def chunk(c, carry):
                v = idx_v[pl.ds(c * ch, ch)]
                for j in range(ch):
                    pltpu.async_copy(emb.at[v[j]], rows_v.at[c * ch + j], sem)
                return carry

            lax.fori_loop(0, bpw // ch, chunk, 0)
            pltpu.make_async_copy(emb.at[pl.ds(0, bpw)], rows_v, sem).wait()
            pltpu.sync_copy(rows_v, out.at[pl.ds(base, bpw)])

        one_table(uemb, uids, out_u)
        one_table(bemb, bids, out_b)

    return gather2


def _mlp_body(xu_ref, xb_ref, w1u_ref, w1b_ref, b1_ref, w2_ref, b2_ref,
              w3_ref, b3_ref, out_ref):
    dn = (((1,), (1,)), ((), ()))
    h = lax.dot_general(xu_ref[...], w1u_ref[...], dn,
                        preferred_element_type=jnp.float32)
    h = h + lax.dot_general(xb_ref[...], w1b_ref[...], dn,
                            preferred_element_type=jnp.float32)
    h = jnp.maximum(h + b1_ref[...], 0.0)
    h = lax.dot_general(h, w2_ref[...], dn, preferred_element_type=jnp.float32)
    h = jnp.maximum(h + b2_ref[...], 0.0)
    out = jnp.sum(h * w3_ref[...], axis=1, keepdims=True)
    out_ref[...] = out + b3_ref[0, 0]


def _mlp(xu, xb, W1u, W1b, b1, W2, b2, W3, b3, blk=2048):
    B, D = xu.shape
    H1 = W1u.shape[0]
    H2 = W2.shape[0]
    grid = (B // blk,)
    full = lambda shape: pl.BlockSpec(shape, lambda i: (0, 0))
    return pl.pallas_call(
        _mlp_body,
        grid=grid,
        in_specs=[
            pl.BlockSpec((blk, D), lambda i: (i, 0)),
            pl.BlockSpec((blk, D), lambda i: (i, 0)),
            full((H1, D)),
            full((H1, D)),
            full((1, H1)),
            full((H2, H1)),
            full((1, H2)),
            full((1, H2)),
            full((1, 1)),
        ],
        out_specs=pl.BlockSpec((blk, 1), lambda i: (i, 0)),
        out_shape=jax.ShapeDtypeStruct((B, 1), jnp.float32),
    )(xu, xb, W1u, W1b, b1, W2, b2, W3, b3)


def kernel(user_ids, book_ids, user_emb, book_emb, W1, b1, W2, b2, W3, b3):
    B = user_ids.shape[0]
    D = user_emb.shape[1]
    xu, xb = _make_gather2(B, D)(user_emb, user_ids.astype(jnp.int32),
                                 book_emb, book_ids.astype(jnp.int32))
    out = _mlp(xu, xb, W1[:, :D], W1[:, D:], b1.reshape(1, -1),
               W2, b2.reshape(1, -1), W3, b3.reshape(1, 1))
    return out.reshape(B)
